# baseline (device time: 77582 ns/iter reference)
import jax
import jax.numpy as jnp
from jax import lax
from jax.experimental import pallas as pl
from jax.experimental.pallas import tpu as pltpu

N_DEV = 32
LOG2 = 5
B, Sq, Hq, Dh = 2, 512, 8, 64
HD = Hq * Dh
SKV = 512
DMODEL = 768
OWN = Sq // N_DEV
NBLK = 8
NRES = 4
GRP = 2 * 64

SIZES = [Sq >> (k + 1) for k in range(LOG2)]


def _perm_lo(blk: int) -> int:
    return 128 * (blk % NRES) + 64 * (blk // NRES)


def kernel(x, Wq, K_ext, V_ext, Wo):
    K2 = K_ext.reshape(B, SKV, HD)
    V2 = V_ext.reshape(B, SKV, HD)

    def body(x_ref, wq_ref, k_ref, v_ref, wo_ref, out_ref,
             o_acc, l_acc, xp, kp, vp,
             or0, or1, or2, or3, or4, lr0, lr1, lr2, lr3, lr4,
             rs_ssem, rs_rsem, l_ssem, l_rsem, ag_ssem, ag_rsem,
             o_final, lt_buf, exit_sem):
        my = lax.axis_index("i")
        o_recv = [or0, or1, or2, or3, or4]
        l_recv = [lr0, lr1, lr2, lr3, lr4]

        for b in range(B):
            for blk in range(NBLK):
                dst = _perm_lo(blk)
                src = 64 * blk
                xp[b, dst:dst + 64, :] = x_ref[
                    b, src:src + 64, :].astype(jnp.bfloat16)
                kp[b, dst:dst + 64, :] = k_ref[
                    b, src:src + 64, :].astype(jnp.bfloat16)
                vp[b, dst:dst + 64, :] = v_ref[
                    b, src:src + 64, :].astype(jnp.bfloat16)

        wq = wq_ref[...].astype(jnp.bfloat16)
        for b in range(B):
            qall = jax.lax.dot(
                xp[b, :, :], wq,
                preferred_element_type=jnp.float32).astype(jnp.bfloat16)
            for h in range(Hq):
                hlo = h * Dh
                for r in range(NRES):
                    g = GRP * r
                    qh = qall[g:g + GRP, hlo:hlo + Dh]
                    kh = kp[b, g:g + GRP, hlo:hlo + Dh]
                    s = jax.lax.dot_general(
                        qh, kh, (((1,), (1,)), ((), ())),
                        preferred_element_type=jnp.float32)
                    w = jnp.exp(s * 0.125)
                    l_acc[b, h, g:g + GRP] = jnp.sum(w, axis=1)
                    o_acc[b, g:g + GRP, hlo:hlo + Dh] = jax.lax.dot(
                        w.astype(jnp.bfloat16), vp[b, g:g + GRP, hlo:hlo + Dh],
                        preferred_element_type=jnp.float32,
                    ).astype(jnp.bfloat16)

        lo = jnp.int32(0)
        for k in range(LOG2):
            sz = SIZES[k]
            bit = (my >> k) & 1
            partner = my ^ (1 << k)
            keep_lo = lo + bit * sz
            send_lo = lo + (1 - bit) * sz
            ro = pltpu.make_async_remote_copy(
                src_ref=o_acc.at[:, pl.ds(send_lo, sz), :], dst_ref=o_recv[k],
                send_sem=rs_ssem.at[k], recv_sem=rs_rsem.at[k],
                device_id=(partner,), device_id_type=pl.DeviceIdType.MESH)
            rl = pltpu.make_async_remote_copy(
                src_ref=l_acc, dst_ref=l_recv[k],
                send_sem=l_ssem.at[k], recv_sem=l_rsem.at[k],
                device_id=(partner,), device_id_type=pl.DeviceIdType.MESH)
            ro.start()
            rl.start()
            ro.wait()
            rl.wait()
            o_acc[:, pl.ds(keep_lo, sz), :] = (
                o_acc[:, pl.ds(keep_lo, sz), :] + o_recv[k][:, :, :])
            l_acc[...] = l_acc[...] + l_recv[k][:, :, :]
            lo = keep_lo

        for b in range(B):
            lt_buf[b, :, :] = l_acc[b, :, :].T
            lb = lt_buf[b, pl.ds(lo, OWN), :]
            lfull = jnp.broadcast_to(
                lb[:, :, None], (OWN, Hq, Dh)).reshape(OWN, HD)
            o_final[b, pl.ds(lo, OWN), :] = (
                o_acc[b, pl.ds(lo, OWN), :].astype(jnp.float32)
                / lfull).astype(jnp.bfloat16)

        r_lo = lo
        for k in reversed(range(LOG2)):
            sz = SIZES[k]
            partner = my ^ (1 << k)
            ag = pltpu.make_async_remote_copy(
                src_ref=o_final.at[:, pl.ds(r_lo, sz), :],
                dst_ref=o_final.at[:, pl.ds(r_lo, sz), :],
                send_sem=ag_ssem.at[k], recv_sem=ag_rsem.at[k],
                device_id=(partner,), device_id_type=pl.DeviceIdType.MESH)
            ag.start()
            ag.wait()
            r_lo = r_lo - ((my >> k) & 1) * sz

        wo = wo_ref[...].astype(jnp.bfloat16)
        for b in range(B):
            outp = jax.lax.dot(
                o_final[b, :, :], wo, preferred_element_type=jnp.float32)
            for blk in range(NBLK):
                src = _perm_lo(blk)
                out_ref[b, 64 * blk:64 * blk + 64, :] = outp[src:src + 64, :]

        for k in range(LOG2):
            pl.semaphore_signal(
                exit_sem, inc=1,
                device_id=(my ^ (1 << k),),
                device_id_type=pl.DeviceIdType.MESH)
        pl.semaphore_wait(exit_sem, LOG2)

    return pl.pallas_call(
        body,
        out_shape=jax.ShapeDtypeStruct((B, Sq, DMODEL), jnp.float32),
        in_specs=[pl.BlockSpec(memory_space=pltpu.VMEM)] * 5,
        out_specs=pl.BlockSpec(memory_space=pltpu.VMEM),
        scratch_shapes=[
            pltpu.VMEM((B, Sq, HD), jnp.bfloat16),
            pltpu.VMEM((B, Hq, Sq), jnp.float32),
            pltpu.VMEM((B, Sq, DMODEL), jnp.bfloat16),
            pltpu.VMEM((B, SKV, HD), jnp.bfloat16),
            pltpu.VMEM((B, SKV, HD), jnp.bfloat16),
            *[pltpu.VMEM((B, s, HD), jnp.bfloat16) for s in SIZES],
            *[pltpu.VMEM((B, Hq, Sq), jnp.float32) for _ in SIZES],
            pltpu.SemaphoreType.DMA((LOG2,)),
            pltpu.SemaphoreType.DMA((LOG2,)),
            pltpu.SemaphoreType.DMA((LOG2,)),
            pltpu.SemaphoreType.DMA((LOG2,)),
            pltpu.SemaphoreType.DMA((LOG2,)),
            pltpu.SemaphoreType.DMA((LOG2,)),
            pltpu.VMEM((B, Sq, HD), jnp.bfloat16),
            pltpu.VMEM((B, Sq, Hq), jnp.float32),
            pltpu.SemaphoreType.REGULAR,
        ],
    )(x, Wq, K2, V2, Wo)


# device time: 67141 ns/iter; 1.1555x vs baseline; 1.1555x over previous
import jax
import jax.numpy as jnp
from jax import lax
from jax.experimental import pallas as pl
from jax.experimental.pallas import tpu as pltpu

N_DEV = 32
B, Sq, Hq, Dh = 2, 512, 8, 64
HD = Hq * Dh
SKV = 512
DMODEL = 768
OWN = Sq // N_DEV
NBLK = 8
NRES = 4
GRP = 128


def _perm_lo(blk: int) -> int:
    return 128 * (blk % NRES) + 64 * (blk // NRES)


def kernel(x, Wq, K_ext, V_ext, Wo):
    K2 = K_ext.reshape(B, SKV, HD)
    V2 = V_ext.reshape(B, SKV, HD)

    def body(x_ref, wq_ref, k_ref, v_ref, wo_ref, out_ref,
             o_acc, l_acc, xp, kp, vp,
             orx, lrx, ory, lry, orz, lrz,
             sx_s, sx_r, sy_s, sy_r, sy_ls, sy_lr,
             sz_s, sz_r, sz_ls, sz_lr,
             agz_s, agz_r, agy_s, agy_r, agx_s, agx_r,
             o_final, lt_buf, exit_sem):
        my = lax.axis_index("i")
        yi = (my >> 1) & 3
        zi = (my >> 3) & 3
        xpartner = my ^ 1

        def y_peer(d):
            return (my & -7) | ((((my >> 1) + d) & 3) << 1)

        def z_peer(d):
            return (my & -25) | ((((my >> 3) + d) & 3) << 3)

        peers = [xpartner] + [y_peer(d) for d in (1, 2, 3)] \
            + [z_peer(d) for d in (1, 2, 3)]

        barrier_sem = pltpu.get_barrier_semaphore()
        for p in peers:
            pl.semaphore_signal(barrier_sem, inc=1, device_id=(p,),
                                device_id_type=pl.DeviceIdType.MESH)
        pl.semaphore_wait(barrier_sem, 7)

        for b in range(B):
            for blk in range(NBLK):
                dst = _perm_lo(blk)
                src = 64 * blk
                xp[b, dst:dst + 64, :] = x_ref[
                    b, src:src + 64, :].astype(jnp.bfloat16)
                kp[b, dst:dst + 64, :] = k_ref[
                    b, src:src + 64, :].astype(jnp.bfloat16)
                vp[b, dst:dst + 64, :] = v_ref[
                    b, src:src + 64, :].astype(jnp.bfloat16)

        wq = wq_ref[...].astype(jnp.bfloat16)
        for b in range(B):
            qall = jax.lax.dot(
                xp[b, :, :], wq,
                preferred_element_type=jnp.float32).astype(jnp.bfloat16)
            for h in range(Hq):
                hlo = h * Dh
                for r in range(NRES):
                    g = GRP * r
                    qh = qall[g:g + GRP, hlo:hlo + Dh]
                    kh = kp[b, g:g + GRP, hlo:hlo + Dh]
                    s = jax.lax.dot_general(
                        qh, kh, (((1,), (1,)), ((), ())),
                        preferred_element_type=jnp.float32)
                    w = jnp.exp(s * 0.125)
                    l_acc[b, h, g:g + GRP] = jnp.sum(w, axis=1)
                    o_acc[b, g:g + GRP, hlo:hlo + Dh] = jax.lax.dot(
                        w.astype(jnp.bfloat16), vp[b, g:g + GRP, hlo:hlo + Dh],
                        preferred_element_type=jnp.float32,
                    ).astype(jnp.bfloat16)

        bit = my & 1
        keep_lo = bit * 256
        send_lo = (1 - bit) * 256
        rox = pltpu.make_async_remote_copy(
            src_ref=o_acc.at[:, pl.ds(send_lo, 256), :], dst_ref=orx,
            send_sem=sx_s.at[0], recv_sem=sx_r.at[0],
            device_id=(xpartner,), device_id_type=pl.DeviceIdType.MESH)
        rlx = pltpu.make_async_remote_copy(
            src_ref=l_acc, dst_ref=lrx,
            send_sem=sx_s.at[1], recv_sem=sx_r.at[1],
            device_id=(xpartner,), device_id_type=pl.DeviceIdType.MESH)
        rox.start()
        rlx.start()
        rox.wait()
        rlx.wait()
        o_acc[:, pl.ds(keep_lo, 256), :] = (
            o_acc[:, pl.ds(keep_lo, 256), :] + orx[:, :, :])
        l_acc[...] = l_acc[...] + lrx[:, :, :]
        lo = keep_lo

        ys = []
        for d in (1, 2, 3):
            pyi = (yi + d) & 3
            ro = pltpu.make_async_remote_copy(
                src_ref=o_acc.at[:, pl.ds(lo + 64 * pyi, 64), :],
                dst_ref=ory.at[3 - d],
                send_sem=sy_s.at[d - 1], recv_sem=sy_r.at[3 - d],
                device_id=(y_peer(d),), device_id_type=pl.DeviceIdType.MESH)
            rl = pltpu.make_async_remote_copy(
                src_ref=l_acc, dst_ref=lry.at[3 - d],
                send_sem=sy_ls.at[d - 1], recv_sem=sy_lr.at[3 - d],
                device_id=(y_peer(d),), device_id_type=pl.DeviceIdType.MESH)
            ro.start()
            rl.start()
            ys.append((ro, rl))
        for ro, rl in ys:
            ro.wait()
            rl.wait()
        lo = lo + 64 * yi
        o_acc[:, pl.ds(lo, 64), :] = (
            o_acc[:, pl.ds(lo, 64), :]
            + ory[0, :, :, :] + ory[1, :, :, :] + ory[2, :, :, :])
        l_acc[...] = (l_acc[...]
                      + lry[0, :, :, :] + lry[1, :, :, :] + lry[2, :, :, :])

        zs = []
        for d in (1, 2, 3):
            pzi = (zi + d) & 3
            ro = pltpu.make_async_remote_copy(
                src_ref=o_acc.at[:, pl.ds(lo + 16 * pzi, 16), :],
                dst_ref=orz.at[3 - d],
                send_sem=sz_s.at[d - 1], recv_sem=sz_r.at[3 - d],
                device_id=(z_peer(d),), device_id_type=pl.DeviceIdType.MESH)
            rl = pltpu.make_async_remote_copy(
                src_ref=l_acc, dst_ref=lrz.at[3 - d],
                send_sem=sz_ls.at[d - 1], recv_sem=sz_lr.at[3 - d],
                device_id=(z_peer(d),), device_id_type=pl.DeviceIdType.MESH)
            ro.start()
            rl.start()
            zs.append((ro, rl))
        for ro, rl in zs:
            ro.wait()
            rl.wait()
        lo = lo + 16 * zi
        o_acc[:, pl.ds(lo, OWN), :] = (
            o_acc[:, pl.ds(lo, OWN), :]
            + orz[0, :, :, :] + orz[1, :, :, :] + orz[2, :, :, :])
        l_acc[...] = (l_acc[...]
                      + lrz[0, :, :, :] + lrz[1, :, :, :] + lrz[2, :, :, :])

        for b in range(B):
            lt_buf[b, :, :] = l_acc[b, :, :].T
            lb = lt_buf[b, pl.ds(lo, OWN), :]
            lfull = jnp.broadcast_to(
                lb[:, :, None], (OWN, Hq, Dh)).reshape(OWN, HD)
            o_final[b, pl.ds(lo, OWN), :] = (
                o_acc[b, pl.ds(lo, OWN), :].astype(jnp.float32)
                / lfull).astype(jnp.bfloat16)

        ags = []
        for d in (1, 2, 3):
            ag = pltpu.make_async_remote_copy(
                src_ref=o_final.at[:, pl.ds(lo, OWN), :],
                dst_ref=o_final.at[:, pl.ds(lo, OWN), :],
                send_sem=agz_s.at[d - 1], recv_sem=agz_r.at[3 - d],
                device_id=(z_peer(d),), device_id_type=pl.DeviceIdType.MESH)
            ag.start()
            ags.append(ag)
        for ag in ags:
            ag.wait_send()
        for s in range(3):
            rr = pltpu.make_async_remote_copy(
                src_ref=o_final.at[:, pl.ds(lo, OWN), :],
                dst_ref=o_final.at[:, pl.ds(lo, OWN), :],
                send_sem=agz_s.at[s], recv_sem=agz_r.at[s],
                device_id=(my,), device_id_type=pl.DeviceIdType.MESH)
            rr.wait_recv()
        lo = lo - 16 * zi

        ags = []
        for d in (1, 2, 3):
            ag = pltpu.make_async_remote_copy(
                src_ref=o_final.at[:, pl.ds(lo, 64), :],
                dst_ref=o_final.at[:, pl.ds(lo, 64), :],
                send_sem=agy_s.at[d - 1], recv_sem=agy_r.at[3 - d],
                device_id=(y_peer(d),), device_id_type=pl.DeviceIdType.MESH)
            ag.start()
            ags.append(ag)
        for ag in ags:
            ag.wait_send()
        for s in range(3):
            rr = pltpu.make_async_remote_copy(
                src_ref=o_final.at[:, pl.ds(lo, 64), :],
                dst_ref=o_final.at[:, pl.ds(lo, 64), :],
                send_sem=agy_s.at[s], recv_sem=agy_r.at[s],
                device_id=(my,), device_id_type=pl.DeviceIdType.MESH)
            rr.wait_recv()
        lo = lo - 64 * yi

        agx = pltpu.make_async_remote_copy(
            src_ref=o_final.at[:, pl.ds(lo, 256), :],
            dst_ref=o_final.at[:, pl.ds(lo, 256), :],
            send_sem=agx_s.at[0], recv_sem=agx_r.at[0],
            device_id=(xpartner,), device_id_type=pl.DeviceIdType.MESH)
        agx.start()
        agx.wait()

        wo = wo_ref[...].astype(jnp.bfloat16)
        for b in range(B):
            outp = jax.lax.dot(
                o_final[b, :, :], wo, preferred_element_type=jnp.float32)
            for blk in range(NBLK):
                src = _perm_lo(blk)
                out_ref[b, 64 * blk:64 * blk + 64, :] = outp[src:src + 64, :]

        for p in peers:
            pl.semaphore_signal(exit_sem, inc=1, device_id=(p,),
                                device_id_type=pl.DeviceIdType.MESH)
        pl.semaphore_wait(exit_sem, 7)

    return pl.pallas_call(
        body,
        out_shape=jax.ShapeDtypeStruct((B, Sq, DMODEL), jnp.float32),
        in_specs=[pl.BlockSpec(memory_space=pltpu.VMEM)] * 5,
        out_specs=pl.BlockSpec(memory_space=pltpu.VMEM),
        scratch_shapes=[
            pltpu.VMEM((B, Sq, HD), jnp.bfloat16),
            pltpu.VMEM((B, Hq, Sq), jnp.float32),
            pltpu.VMEM((B, Sq, DMODEL), jnp.bfloat16),
            pltpu.VMEM((B, SKV, HD), jnp.bfloat16),
            pltpu.VMEM((B, SKV, HD), jnp.bfloat16),
            pltpu.VMEM((B, 256, HD), jnp.bfloat16),
            pltpu.VMEM((B, Hq, Sq), jnp.float32),
            pltpu.VMEM((3, B, 64, HD), jnp.bfloat16),
            pltpu.VMEM((3, B, Hq, Sq), jnp.float32),
            pltpu.VMEM((3, B, OWN, HD), jnp.bfloat16),
            pltpu.VMEM((3, B, Hq, Sq), jnp.float32),
            pltpu.SemaphoreType.DMA((2,)),
            pltpu.SemaphoreType.DMA((2,)),
            pltpu.SemaphoreType.DMA((3,)),
            pltpu.SemaphoreType.DMA((3,)),
            pltpu.SemaphoreType.DMA((3,)),
            pltpu.SemaphoreType.DMA((3,)),
            pltpu.SemaphoreType.DMA((3,)),
            pltpu.SemaphoreType.DMA((3,)),
            pltpu.SemaphoreType.DMA((3,)),
            pltpu.SemaphoreType.DMA((3,)),
            pltpu.SemaphoreType.DMA((3,)),
            pltpu.SemaphoreType.DMA((3,)),
            pltpu.SemaphoreType.DMA((3,)),
            pltpu.SemaphoreType.DMA((3,)),
            pltpu.SemaphoreType.DMA((1,)),
            pltpu.SemaphoreType.DMA((1,)),
            pltpu.VMEM((B, Sq, HD), jnp.bfloat16),
            pltpu.VMEM((B, Sq, Hq), jnp.float32),
            pltpu.SemaphoreType.REGULAR,
        ],
        compiler_params=pltpu.CompilerParams(collective_id=0),
    )(x, Wq, K2, V2, Wo)


# device time: 62991 ns/iter; 1.2316x vs baseline; 1.0659x over previous
import jax
import jax.numpy as jnp
from jax import lax
from jax.experimental import pallas as pl
from jax.experimental.pallas import tpu as pltpu

N_DEV = 32
B, Sq, Hq, Dh = 2, 512, 8, 64
HD = Hq * Dh
SKV = 512
DMODEL = 768
OWN = Sq // N_DEV
NBLK = 8
NRES = 4
GRP = 128
NYZ = 16


def _perm_lo(blk: int) -> int:
    return 128 * (blk % NRES) + 64 * (blk // NRES)


def kernel(x, Wq, K_ext, V_ext, Wo):
    K2 = K_ext.reshape(B, SKV, HD)
    V2 = V_ext.reshape(B, SKV, HD)

    def body(x_ref, wq_ref, k_ref, v_ref, wo_ref, out_ref,
             o_acc, l_acc, l_tmp, xp, kp, vp,
             orx, lrx, oryz, lryz,
             sx_s, sx_r, syz_s, syz_r, syz_ls, syz_lr,
             agyz_s, agyz_r, agx_s, agx_r,
             o_final, lt_buf, exit_sem):
        my = lax.axis_index("i")
        bit = my & 1
        yi = (my >> 1) & 3
        zi = (my >> 3) & 3
        t = (zi << 2) | yi
        xpartner = my ^ 1

        def yz_peer(d):
            pt = (t + d) & 15
            return bit | ((pt & 3) << 1) | ((pt >> 2) << 3)

        peers = [xpartner] + [yz_peer(d) for d in range(1, NYZ)]

        barrier_sem = pltpu.get_barrier_semaphore()
        for p in peers:
            pl.semaphore_signal(barrier_sem, inc=1, device_id=(p,),
                                device_id_type=pl.DeviceIdType.MESH)
        pl.semaphore_wait(barrier_sem, NYZ)

        for b in range(B):
            for blk in range(NBLK):
                dst = _perm_lo(blk)
                src = 64 * blk
                xp[b, dst:dst + 64, :] = x_ref[
                    b, src:src + 64, :].astype(jnp.bfloat16)
                kp[b, dst:dst + 64, :] = k_ref[
                    b, src:src + 64, :].astype(jnp.bfloat16)
                vp[b, dst:dst + 64, :] = v_ref[
                    b, src:src + 64, :].astype(jnp.bfloat16)

        keep_lo = bit * 256
        send_lo = (1 - bit) * 256
        rox = pltpu.make_async_remote_copy(
            src_ref=o_acc.at[:, pl.ds(send_lo, 256), :], dst_ref=orx,
            send_sem=sx_s.at[0], recv_sem=sx_r.at[0],
            device_id=(xpartner,), device_id_type=pl.DeviceIdType.MESH)
        rlx = pltpu.make_async_remote_copy(
            src_ref=l_acc, dst_ref=lrx,
            send_sem=sx_s.at[1], recv_sem=sx_r.at[1],
            device_id=(xpartner,), device_id_type=pl.DeviceIdType.MESH)

        wq = wq_ref[...].astype(jnp.bfloat16)
        for p, base in enumerate([send_lo, keep_lo]):
            for b in range(B):
                qhalf = jax.lax.dot(
                    xp[b, pl.ds(base, 256), :], wq,
                    preferred_element_type=jnp.float32).astype(jnp.bfloat16)
                for h in range(Hq):
                    hlo = h * Dh
                    for j in range(2):
                        grow = base + GRP * j
                        gl = GRP * j
                        qh = qhalf[gl:gl + GRP, hlo:hlo + Dh]
                        kh = kp[b, pl.ds(grow, GRP), hlo:hlo + Dh]
                        s = jax.lax.dot_general(
                            qh, kh, (((1,), (1,)), ((), ())),
                            preferred_element_type=jnp.float32)
                        w = jnp.exp(s * 0.125)
                        l_tmp[b, p, h, gl:gl + GRP] = jnp.sum(w, axis=1)
                        o_acc[b, pl.ds(grow, GRP), hlo:hlo + Dh] = jax.lax.dot(
                            w.astype(jnp.bfloat16),
                            vp[b, pl.ds(grow, GRP), hlo:hlo + Dh],
                            preferred_element_type=jnp.float32,
                        ).astype(jnp.bfloat16)
            if p == 0:
                rox.start()

        @pl.when(bit == 0)
        def _():
            l_acc[:, :, 256:512] = l_tmp[:, 0, :, :]
            l_acc[:, :, 0:256] = l_tmp[:, 1, :, :]

        @pl.when(bit == 1)
        def _():
            l_acc[:, :, 0:256] = l_tmp[:, 0, :, :]
            l_acc[:, :, 256:512] = l_tmp[:, 1, :, :]

        rlx.start()

        rox.wait()
        rlx.wait()
        o_acc[:, pl.ds(keep_lo, 256), :] = (
            o_acc[:, pl.ds(keep_lo, 256), :] + orx[:, :, :])
        l_acc[...] = l_acc[...] + lrx[:, :, :]
        lo = keep_lo

        rs = []
        for d in range(1, NYZ):
            pt = (t + d) & 15
            ro = pltpu.make_async_remote_copy(
                src_ref=o_acc.at[:, pl.ds(lo + OWN * pt, OWN), :],
                dst_ref=oryz.at[NYZ - 1 - d],
                send_sem=syz_s.at[d - 1], recv_sem=syz_r.at[NYZ - 1 - d],
                device_id=(yz_peer(d),), device_id_type=pl.DeviceIdType.MESH)
            rl = pltpu.make_async_remote_copy(
                src_ref=l_acc, dst_ref=lryz.at[NYZ - 1 - d],
                send_sem=syz_ls.at[d - 1], recv_sem=syz_lr.at[NYZ - 1 - d],
                device_id=(yz_peer(d),), device_id_type=pl.DeviceIdType.MESH)
            ro.start()
            rl.start()
            rs.append((ro, rl))
        for ro, rl in rs:
            ro.wait()
            rl.wait()
        lo = lo + OWN * t
        osum = oryz[0, :, :, :]
        lsum = lryz[0, :, :, :]
        for s in range(1, NYZ - 1):
            osum = osum + oryz[s, :, :, :]
            lsum = lsum + lryz[s, :, :, :]
        o_acc[:, pl.ds(lo, OWN), :] = o_acc[:, pl.ds(lo, OWN), :] + osum
        l_acc[...] = l_acc[...] + lsum

        for b in range(B):
            lt_buf[b, :, :] = l_acc[b, :, :].T
            lb = lt_buf[b, pl.ds(lo, OWN), :]
            lfull = jnp.broadcast_to(
                lb[:, :, None], (OWN, Hq, Dh)).reshape(OWN, HD)
            o_final[b, pl.ds(lo, OWN), :] = (
                o_acc[b, pl.ds(lo, OWN), :].astype(jnp.float32)
                / lfull).astype(jnp.bfloat16)

        ags = []
        for d in range(1, NYZ):
            ag = pltpu.make_async_remote_copy(
                src_ref=o_final.at[:, pl.ds(lo, OWN), :],
                dst_ref=o_final.at[:, pl.ds(lo, OWN), :],
                send_sem=agyz_s.at[d - 1], recv_sem=agyz_r.at[NYZ - 1 - d],
                device_id=(yz_peer(d),), device_id_type=pl.DeviceIdType.MESH)
            ag.start()
            ags.append(ag)
        for ag in ags:
            ag.wait_send()
        for s in range(NYZ - 1):
            rr = pltpu.make_async_remote_copy(
                src_ref=o_final.at[:, pl.ds(lo, OWN), :],
                dst_ref=o_final.at[:, pl.ds(lo, OWN), :],
                send_sem=agyz_s.at[s], recv_sem=agyz_r.at[s],
                device_id=(my,), device_id_type=pl.DeviceIdType.MESH)
            rr.wait_recv()
        lo = lo - OWN * t

        agx = pltpu.make_async_remote_copy(
            src_ref=o_final.at[:, pl.ds(lo, 256), :],
            dst_ref=o_final.at[:, pl.ds(lo, 256), :],
            send_sem=agx_s.at[0], recv_sem=agx_r.at[0],
            device_id=(xpartner,), device_id_type=pl.DeviceIdType.MESH)
        agx.start()

        wo = wo_ref[...].astype(jnp.bfloat16)

        def wo_half(half_lo):
            for b in range(B):
                outp = jax.lax.dot(
                    o_final[b, pl.ds(half_lo, 256), :], wo,
                    preferred_element_type=jnp.float32)
                for c in range(4):
                    g = (half_lo + 64 * c) >> 7
                    dst = 64 * g + 256 * (c & 1)
                    out_ref[b, pl.ds(dst, 64), :] = outp[64 * c:64 * c + 64, :]

        wo_half(lo)
        agx.wait()
        wo_half(256 - lo)

        for p in peers:
            pl.semaphore_signal(exit_sem, inc=1, device_id=(p,),
                                device_id_type=pl.DeviceIdType.MESH)
        pl.semaphore_wait(exit_sem, NYZ)

    return pl.pallas_call(
        body,
        out_shape=jax.ShapeDtypeStruct((B, Sq, DMODEL), jnp.float32),
        in_specs=[pl.BlockSpec(memory_space=pltpu.VMEM)] * 5,
        out_specs=pl.BlockSpec(memory_space=pltpu.VMEM),
        scratch_shapes=[
            pltpu.VMEM((B, Sq, HD), jnp.bfloat16),
            pltpu.VMEM((B, Hq, Sq), jnp.float32),
            pltpu.VMEM((B, 2, Hq, 256), jnp.float32),
            pltpu.VMEM((B, Sq, DMODEL), jnp.bfloat16),
            pltpu.VMEM((B, SKV, HD), jnp.bfloat16),
            pltpu.VMEM((B, SKV, HD), jnp.bfloat16),
            pltpu.VMEM((B, 256, HD), jnp.bfloat16),
            pltpu.VMEM((B, Hq, Sq), jnp.float32),
            pltpu.VMEM((NYZ - 1, B, OWN, HD), jnp.bfloat16),
            pltpu.VMEM((NYZ - 1, B, Hq, Sq), jnp.float32),
            pltpu.SemaphoreType.DMA((2,)),
            pltpu.SemaphoreType.DMA((2,)),
            pltpu.SemaphoreType.DMA((NYZ - 1,)),
            pltpu.SemaphoreType.DMA((NYZ - 1,)),
            pltpu.SemaphoreType.DMA((NYZ - 1,)),
            pltpu.SemaphoreType.DMA((NYZ - 1,)),
            pltpu.SemaphoreType.DMA((NYZ - 1,)),
            pltpu.SemaphoreType.DMA((NYZ - 1,)),
            pltpu.SemaphoreType.DMA((1,)),
            pltpu.SemaphoreType.DMA((1,)),
            pltpu.VMEM((B, Sq, HD), jnp.bfloat16),
            pltpu.VMEM((B, Sq, Hq), jnp.float32),
            pltpu.SemaphoreType.REGULAR,
        ],
        compiler_params=pltpu.CompilerParams(collective_id=0),
    )(x, Wq, K2, V2, Wo)


# device time: 62797 ns/iter; 1.2354x vs baseline; 1.0031x over previous
import jax
import jax.numpy as jnp
from jax import lax
from jax.experimental import pallas as pl
from jax.experimental.pallas import tpu as pltpu

N_DEV = 32
B, Sq, Hq, Dh = 2, 512, 8, 64
HD = Hq * Dh
SKV = 512
DMODEL = 768
OWN = Sq // N_DEV
NBLK = 8
NRES = 4
GRP = 128
NYZ = 16


def _perm_lo(blk: int) -> int:
    return 128 * (blk % NRES) + 64 * (blk // NRES)


def kernel(x, Wq, K_ext, V_ext, Wo):
    K2 = K_ext.reshape(B, SKV, HD)
    V2 = V_ext.reshape(B, SKV, HD)

    def body(x_ref, wq_ref, k_ref, v_ref, wo_ref, out_ref,
             o_acc, lt_acc, l_tmp, xp, kp, vp,
             orx, lrx, oryz, lryz,
             sx_s, sx_r, syz_s, syz_r, syz_ls, syz_lr,
             agyz_s, agyz_r, agx_s, agx_r,
             o_final, exit_sem):
        my = lax.axis_index("i")
        bit = my & 1
        yi = (my >> 1) & 3
        zi = (my >> 3) & 3
        t = (zi << 2) | yi
        xpartner = my ^ 1

        def yz_peer(d):
            pt = (t + d) & 15
            return bit | ((pt & 3) << 1) | ((pt >> 2) << 3)

        peers = [xpartner] + [yz_peer(d) for d in range(1, NYZ)]

        barrier_sem = pltpu.get_barrier_semaphore()
        for p in peers:
            pl.semaphore_signal(barrier_sem, inc=1, device_id=(p,),
                                device_id_type=pl.DeviceIdType.MESH)
        pl.semaphore_wait(barrier_sem, NYZ)

        for b in range(B):
            for blk in range(NBLK):
                dst = _perm_lo(blk)
                src = 64 * blk
                xp[b, dst:dst + 64, :] = x_ref[
                    b, src:src + 64, :].astype(jnp.bfloat16)
                kp[b, dst:dst + 64, :] = k_ref[
                    b, src:src + 64, :].astype(jnp.bfloat16)
                vp[b, dst:dst + 64, :] = v_ref[
                    b, src:src + 64, :].astype(jnp.bfloat16)

        keep_lo = bit * 256
        send_lo = (1 - bit) * 256
        rox = pltpu.make_async_remote_copy(
            src_ref=o_acc.at[:, pl.ds(send_lo, 256), :], dst_ref=orx,
            send_sem=sx_s.at[0], recv_sem=sx_r.at[0],
            device_id=(xpartner,), device_id_type=pl.DeviceIdType.MESH)
        rlx = pltpu.make_async_remote_copy(
            src_ref=lt_acc.at[:, pl.ds(send_lo, 256), :], dst_ref=lrx,
            send_sem=sx_s.at[1], recv_sem=sx_r.at[1],
            device_id=(xpartner,), device_id_type=pl.DeviceIdType.MESH)

        wq = wq_ref[...].astype(jnp.bfloat16)
        for p, base in enumerate([send_lo, keep_lo]):
            for b in range(B):
                qhalf = jax.lax.dot(
                    xp[b, pl.ds(base, 256), :], wq,
                    preferred_element_type=jnp.float32).astype(jnp.bfloat16)
                for h in range(Hq):
                    hlo = h * Dh
                    for j in range(2):
                        grow = base + GRP * j
                        gl = GRP * j
                        qh = qhalf[gl:gl + GRP, hlo:hlo + Dh]
                        kh = kp[b, pl.ds(grow, GRP), hlo:hlo + Dh]
                        s = jax.lax.dot_general(
                            qh, kh, (((1,), (1,)), ((), ())),
                            preferred_element_type=jnp.float32)
                        w = jnp.exp(s * 0.125)
                        l_tmp[b, p, h, gl:gl + GRP] = jnp.sum(w, axis=1)
                        o_acc[b, pl.ds(grow, GRP), hlo:hlo + Dh] = jax.lax.dot(
                            w.astype(jnp.bfloat16),
                            vp[b, pl.ds(grow, GRP), hlo:hlo + Dh],
                            preferred_element_type=jnp.float32,
                        ).astype(jnp.bfloat16)
            if p == 0:
                rox.start()

        @pl.when(bit == 0)
        def _():
            for b in range(B):
                lt_acc[b, 256:512, :] = l_tmp[b, 0, :, :].T
                lt_acc[b, 0:256, :] = l_tmp[b, 1, :, :].T

        @pl.when(bit == 1)
        def _():
            for b in range(B):
                lt_acc[b, 0:256, :] = l_tmp[b, 0, :, :].T
                lt_acc[b, 256:512, :] = l_tmp[b, 1, :, :].T

        rlx.start()

        rox.wait()
        rlx.wait()
        o_acc[:, pl.ds(keep_lo, 256), :] = (
            o_acc[:, pl.ds(keep_lo, 256), :] + orx[:, :, :])
        lt_acc[:, pl.ds(keep_lo, 256), :] = (
            lt_acc[:, pl.ds(keep_lo, 256), :] + lrx[:, :, :])
        lo = keep_lo

        rs = []
        for d in range(1, NYZ):
            pt = (t + d) & 15
            ro = pltpu.make_async_remote_copy(
                src_ref=o_acc.at[:, pl.ds(lo + OWN * pt, OWN), :],
                dst_ref=oryz.at[NYZ - 1 - d],
                send_sem=syz_s.at[d - 1], recv_sem=syz_r.at[NYZ - 1 - d],
                device_id=(yz_peer(d),), device_id_type=pl.DeviceIdType.MESH)
            rl = pltpu.make_async_remote_copy(
                src_ref=lt_acc.at[:, pl.ds(lo + OWN * pt, OWN), :],
                dst_ref=lryz.at[NYZ - 1 - d],
                send_sem=syz_ls.at[d - 1], recv_sem=syz_lr.at[NYZ - 1 - d],
                device_id=(yz_peer(d),), device_id_type=pl.DeviceIdType.MESH)
            ro.start()
            rl.start()
            rs.append((ro, rl))
        for ro, rl in rs:
            ro.wait()
            rl.wait()
        lo = lo + OWN * t
        osum = oryz[0, :, :, :]
        lsum = lryz[0, :, :, :]
        for s in range(1, NYZ - 1):
            osum = osum + oryz[s, :, :, :]
            lsum = lsum + lryz[s, :, :, :]
        o_acc[:, pl.ds(lo, OWN), :] = o_acc[:, pl.ds(lo, OWN), :] + osum
        lt_acc[:, pl.ds(lo, OWN), :] = lt_acc[:, pl.ds(lo, OWN), :] + lsum

        for b in range(B):
            lb = lt_acc[b, pl.ds(lo, OWN), :]
            lfull = jnp.broadcast_to(
                lb[:, :, None], (OWN, Hq, Dh)).reshape(OWN, HD)
            o_final[b, pl.ds(lo, OWN), :] = (
                o_acc[b, pl.ds(lo, OWN), :].astype(jnp.float32)
                / lfull).astype(jnp.bfloat16)

        ags = []
        for d in range(1, NYZ):
            ag = pltpu.make_async_remote_copy(
                src_ref=o_final.at[:, pl.ds(lo, OWN), :],
                dst_ref=o_final.at[:, pl.ds(lo, OWN), :],
                send_sem=agyz_s.at[d - 1], recv_sem=agyz_r.at[NYZ - 1 - d],
                device_id=(yz_peer(d),), device_id_type=pl.DeviceIdType.MESH)
            ag.start()
            ags.append(ag)
        for ag in ags:
            ag.wait_send()
        for s in range(NYZ - 1):
            rr = pltpu.make_async_remote_copy(
                src_ref=o_final.at[:, pl.ds(lo, OWN), :],
                dst_ref=o_final.at[:, pl.ds(lo, OWN), :],
                send_sem=agyz_s.at[s], recv_sem=agyz_r.at[s],
                device_id=(my,), device_id_type=pl.DeviceIdType.MESH)
            rr.wait_recv()
        lo = lo - OWN * t

        agx = pltpu.make_async_remote_copy(
            src_ref=o_final.at[:, pl.ds(lo, 256), :],
            dst_ref=o_final.at[:, pl.ds(lo, 256), :],
            send_sem=agx_s.at[0], recv_sem=agx_r.at[0],
            device_id=(xpartner,), device_id_type=pl.DeviceIdType.MESH)
        agx.start()

        wo = wo_ref[...].astype(jnp.bfloat16)

        def wo_half(half_lo):
            for b in range(B):
                outp = jax.lax.dot(
                    o_final[b, pl.ds(half_lo, 256), :], wo,
                    preferred_element_type=jnp.float32)
                for c in range(4):
                    g = (half_lo + 64 * c) >> 7
                    dst = 64 * g + 256 * (c & 1)
                    out_ref[b, pl.ds(dst, 64), :] = outp[64 * c:64 * c + 64, :]

        wo_half(lo)
        agx.wait()
        wo_half(256 - lo)

        for p in peers:
            pl.semaphore_signal(exit_sem, inc=1, device_id=(p,),
                                device_id_type=pl.DeviceIdType.MESH)
        pl.semaphore_wait(exit_sem, NYZ)

    return pl.pallas_call(
        body,
        out_shape=jax.ShapeDtypeStruct((B, Sq, DMODEL), jnp.float32),
        in_specs=[pl.BlockSpec(memory_space=pltpu.VMEM)] * 5,
        out_specs=pl.BlockSpec(memory_space=pltpu.VMEM),
        scratch_shapes=[
            pltpu.VMEM((B, Sq, HD), jnp.bfloat16),
            pltpu.VMEM((B, Sq, Hq), jnp.float32),
            pltpu.VMEM((B, 2, Hq, 256), jnp.float32),
            pltpu.VMEM((B, Sq, DMODEL), jnp.bfloat16),
            pltpu.VMEM((B, SKV, HD), jnp.bfloat16),
            pltpu.VMEM((B, SKV, HD), jnp.bfloat16),
            pltpu.VMEM((B, 256, HD), jnp.bfloat16),
            pltpu.VMEM((B, 256, Hq), jnp.float32),
            pltpu.VMEM((NYZ - 1, B, OWN, HD), jnp.bfloat16),
            pltpu.VMEM((NYZ - 1, B, OWN, Hq), jnp.float32),
            pltpu.SemaphoreType.DMA((2,)),
            pltpu.SemaphoreType.DMA((2,)),
            pltpu.SemaphoreType.DMA((NYZ - 1,)),
            pltpu.SemaphoreType.DMA((NYZ - 1,)),
            pltpu.SemaphoreType.DMA((NYZ - 1,)),
            pltpu.SemaphoreType.DMA((NYZ - 1,)),
            pltpu.SemaphoreType.DMA((NYZ - 1,)),
            pltpu.SemaphoreType.DMA((NYZ - 1,)),
            pltpu.SemaphoreType.DMA((1,)),
            pltpu.SemaphoreType.DMA((1,)),
            pltpu.VMEM((B, Sq, HD), jnp.bfloat16),
            pltpu.SemaphoreType.REGULAR,
        ],
        compiler_params=pltpu.CompilerParams(collective_id=0),
    )(x, Wq, K2, V2, Wo)


# device time: 59376 ns/iter; 1.3066x vs baseline; 1.0576x over previous
import jax
import jax.numpy as jnp
from jax import lax
from jax.experimental import pallas as pl
from jax.experimental.pallas import tpu as pltpu

N_DEV = 32
B, Sq, Hq, Dh = 2, 512, 8, 64
HD = Hq * Dh
HDL = HD + 128
SKV = 512
DMODEL = 768
OWN = Sq // N_DEV
NBLK = 8
NRES = 4
GRP = 128
NYZ = 16


def _perm_lo(blk: int) -> int:
    return 128 * (blk % NRES) + 64 * (blk // NRES)


def kernel(x, Wq, K_ext, V_ext, Wo):
    K2 = K_ext.reshape(B, SKV, HD)
    V2 = V_ext.reshape(B, SKV, HD)

    def body(x_ref, wq_ref, k_ref, v_ref, wo_ref, out_ref,
             o_acc, xp, kp, vp, orx, oryz,
             sx_s, sx_r, syz_s, syz_r,
             agyz_s, agyz_r, agx_s, agx_r,
             o_final, exit_sem):
        my = lax.axis_index("i")
        bit = my & 1
        yi = (my >> 1) & 3
        zi = (my >> 3) & 3
        t = (zi << 2) | yi
        xpartner = my ^ 1

        def yz_peer(d):
            pt = (t + d) & 15
            return bit | ((pt & 3) << 1) | ((pt >> 2) << 3)

        peers = [xpartner] + [yz_peer(d) for d in range(1, NYZ)]

        barrier_sem = pltpu.get_barrier_semaphore()
        for p in peers:
            pl.semaphore_signal(barrier_sem, inc=1, device_id=(p,),
                                device_id_type=pl.DeviceIdType.MESH)
        pl.semaphore_wait(barrier_sem, NYZ)

        for b in range(B):
            for blk in range(NBLK):
                dst = _perm_lo(blk)
                src = 64 * blk
                xp[b, dst:dst + 64, :] = x_ref[
                    b, src:src + 64, :].astype(jnp.bfloat16)
                kp[b, dst:dst + 64, :] = k_ref[
                    b, src:src + 64, :].astype(jnp.bfloat16)
                vp[b, dst:dst + 64, :] = v_ref[
                    b, src:src + 64, :].astype(jnp.bfloat16)

        keep_lo = bit * 256
        send_lo = (1 - bit) * 256
        rox = pltpu.make_async_remote_copy(
            src_ref=o_acc.at[:, pl.ds(send_lo, 256), :], dst_ref=orx,
            send_sem=sx_s.at[0], recv_sem=sx_r.at[0],
            device_id=(xpartner,), device_id_type=pl.DeviceIdType.MESH)

        wq = wq_ref[...].astype(jnp.bfloat16)
        for p, base in enumerate([send_lo, keep_lo]):
            for b in range(B):
                qhalf = jax.lax.dot(
                    xp[b, pl.ds(base, 256), :], wq,
                    preferred_element_type=jnp.float32).astype(jnp.bfloat16)
                for j in range(2):
                    grow = base + GRP * j
                    gl = GRP * j
                    lvecs = []
                    for h in range(Hq):
                        hlo = h * Dh
                        qh = qhalf[gl:gl + GRP, hlo:hlo + Dh]
                        kh = kp[b, pl.ds(grow, GRP), hlo:hlo + Dh]
                        s = jax.lax.dot_general(
                            qh, kh, (((1,), (1,)), ((), ())),
                            preferred_element_type=jnp.float32)
                        w = jnp.exp(s * 0.125)
                        lvecs.append(jnp.sum(w, axis=1))
                        o_acc[b, pl.ds(grow, GRP), hlo:hlo + Dh] = jax.lax.dot(
                            w.astype(jnp.bfloat16),
                            vp[b, pl.ds(grow, GRP), hlo:hlo + Dh],
                            preferred_element_type=jnp.float32,
                        ).astype(jnp.bfloat16)
                    lg = jnp.stack(lvecs, axis=-1)
                    o_acc[b, pl.ds(grow, GRP), HD:HD + Hq] = lg.astype(
                        jnp.bfloat16)
            if p == 0:
                rox.start()

        rox.wait()
        o_acc[:, pl.ds(keep_lo, 256), :] = (
            o_acc[:, pl.ds(keep_lo, 256), :].astype(jnp.float32)
            + orx[:, :, :].astype(jnp.float32)).astype(jnp.bfloat16)
        lo = keep_lo

        rs = []
        for d in range(1, NYZ):
            pt = (t + d) & 15
            ro = pltpu.make_async_remote_copy(
                src_ref=o_acc.at[:, pl.ds(lo + OWN * pt, OWN), :],
                dst_ref=oryz.at[NYZ - 1 - d],
                send_sem=syz_s.at[d - 1], recv_sem=syz_r.at[NYZ - 1 - d],
                device_id=(yz_peer(d),), device_id_type=pl.DeviceIdType.MESH)
            ro.start()
            rs.append(ro)
        for ro in rs:
            ro.wait()
        lo = lo + OWN * t
        osum = oryz[0, :, :, :].astype(jnp.float32)
        for s in range(1, NYZ - 1):
            osum = osum + oryz[s, :, :, :].astype(jnp.float32)
        o_acc[:, pl.ds(lo, OWN), :] = (
            o_acc[:, pl.ds(lo, OWN), :].astype(jnp.float32)
            + osum).astype(jnp.bfloat16)

        for b in range(B):
            lb = o_acc[b, pl.ds(lo, OWN), HD:HD + Hq].astype(jnp.float32)
            lfull = jnp.broadcast_to(
                lb[:, :, None], (OWN, Hq, Dh)).reshape(OWN, HD)
            o_final[b, pl.ds(lo, OWN), :] = (
                o_acc[b, pl.ds(lo, OWN), 0:HD].astype(jnp.float32)
                / lfull).astype(jnp.bfloat16)

        ags = []
        for d in range(1, NYZ):
            ag = pltpu.make_async_remote_copy(
                src_ref=o_final.at[:, pl.ds(lo, OWN), :],
                dst_ref=o_final.at[:, pl.ds(lo, OWN), :],
                send_sem=agyz_s.at[d - 1], recv_sem=agyz_r.at[NYZ - 1 - d],
                device_id=(yz_peer(d),), device_id_type=pl.DeviceIdType.MESH)
            ag.start()
            ags.append(ag)
        for ag in ags:
            ag.wait_send()
        for s in range(NYZ - 1):
            rr = pltpu.make_async_remote_copy(
                src_ref=o_final.at[:, pl.ds(lo, OWN), :],
                dst_ref=o_final.at[:, pl.ds(lo, OWN), :],
                send_sem=agyz_s.at[s], recv_sem=agyz_r.at[s],
                device_id=(my,), device_id_type=pl.DeviceIdType.MESH)
            rr.wait_recv()
        lo = lo - OWN * t

        agx = pltpu.make_async_remote_copy(
            src_ref=o_final.at[:, pl.ds(lo, 256), :],
            dst_ref=o_final.at[:, pl.ds(lo, 256), :],
            send_sem=agx_s.at[0], recv_sem=agx_r.at[0],
            device_id=(xpartner,), device_id_type=pl.DeviceIdType.MESH)
        agx.start()

        wo = wo_ref[...].astype(jnp.bfloat16)

        def wo_half(half_lo):
            for b in range(B):
                outp = jax.lax.dot(
                    o_final[b, pl.ds(half_lo, 256), :], wo,
                    preferred_element_type=jnp.float32)
                for c in range(4):
                    g = (half_lo + 64 * c) >> 7
                    dst = 64 * g + 256 * (c & 1)
                    out_ref[b, pl.ds(dst, 64), :] = outp[64 * c:64 * c + 64, :]

        wo_half(lo)
        agx.wait()
        wo_half(256 - lo)

        for p in peers:
            pl.semaphore_signal(exit_sem, inc=1, device_id=(p,),
                                device_id_type=pl.DeviceIdType.MESH)
        pl.semaphore_wait(exit_sem, NYZ)

    return pl.pallas_call(
        body,
        out_shape=jax.ShapeDtypeStruct((B, Sq, DMODEL), jnp.float32),
        in_specs=[pl.BlockSpec(memory_space=pltpu.VMEM)] * 5,
        out_specs=pl.BlockSpec(memory_space=pltpu.VMEM),
        scratch_shapes=[
            pltpu.VMEM((B, Sq, HDL), jnp.bfloat16),
            pltpu.VMEM((B, Sq, DMODEL), jnp.bfloat16),
            pltpu.VMEM((B, SKV, HD), jnp.bfloat16),
            pltpu.VMEM((B, SKV, HD), jnp.bfloat16),
            pltpu.VMEM((B, 256, HDL), jnp.bfloat16),
            pltpu.VMEM((NYZ - 1, B, OWN, HDL), jnp.bfloat16),
            pltpu.SemaphoreType.DMA((1,)),
            pltpu.SemaphoreType.DMA((1,)),
            pltpu.SemaphoreType.DMA((NYZ - 1,)),
            pltpu.SemaphoreType.DMA((NYZ - 1,)),
            pltpu.SemaphoreType.DMA((NYZ - 1,)),
            pltpu.SemaphoreType.DMA((NYZ - 1,)),
            pltpu.SemaphoreType.DMA((1,)),
            pltpu.SemaphoreType.DMA((1,)),
            pltpu.VMEM((B, Sq, HD), jnp.bfloat16),
            pltpu.SemaphoreType.REGULAR,
        ],
        compiler_params=pltpu.CompilerParams(collective_id=0),
    )(x, Wq, K2, V2, Wo)


# device time: 51831 ns/iter; 1.4968x vs baseline; 1.1456x over previous
import jax
import jax.numpy as jnp
from jax import lax
from jax.experimental import pallas as pl
from jax.experimental.pallas import tpu as pltpu

N_DEV = 32
B, Sq, Hq, Dh = 2, 512, 8, 64
HD = Hq * Dh
HDL = HD + 128
SKV = 512
DMODEL = 768
OWN = Sq // N_DEV
NBLK = 8
NRES = 4
GRP = 128
NYZ = 16


def _perm_lo(blk: int) -> int:
    return 128 * (blk % NRES) + 64 * (blk // NRES)


def kernel(x, Wq, K_ext, V_ext, Wo):
    K2 = K_ext.reshape(B, SKV, HD)
    V2 = V_ext.reshape(B, SKV, HD)

    def body(x_ref, wq_ref, k_ref, v_ref, wo_ref, out_ref,
             o_acc, xp, kp, vp, orx, oryz,
             sx_s, sx_r, syz_s, syz_r,
             agyz_s, agyz_r, agx_s, agx_r,
             o_final, exit_sem):
        my = lax.axis_index("i")
        bit = my & 1
        yi = (my >> 1) & 3
        zi = (my >> 3) & 3
        t = (zi << 2) | yi
        xpartner = my ^ 1

        def yz_peer(d):
            pt = (t + d) & 15
            return bit | ((pt & 3) << 1) | ((pt >> 2) << 3)

        peers = [xpartner] + [yz_peer(d) for d in range(1, NYZ)]

        barrier_sem = pltpu.get_barrier_semaphore()
        for p in peers:
            pl.semaphore_signal(barrier_sem, inc=1, device_id=(p,),
                                device_id_type=pl.DeviceIdType.MESH)

        for b in range(B):
            for blk in range(NBLK):
                dst = _perm_lo(blk)
                src = 64 * blk
                xp[b, dst:dst + 64, :] = x_ref[
                    b, src:src + 64, :].astype(jnp.bfloat16)
                kp[b, dst:dst + 64, :] = k_ref[
                    b, src:src + 64, :].astype(jnp.bfloat16)
                vp[b, dst:dst + 64, :] = v_ref[
                    b, src:src + 64, :].astype(jnp.bfloat16)

        keep_lo = bit * 256
        send_lo = (1 - bit) * 256
        rox = pltpu.make_async_remote_copy(
            src_ref=o_acc.at[:, pl.ds(send_lo, 256), :], dst_ref=orx,
            send_sem=sx_s.at[0], recv_sem=sx_r.at[0],
            device_id=(xpartner,), device_id_type=pl.DeviceIdType.MESH)

        wq = wq_ref[...].astype(jnp.bfloat16)
        for p, base in enumerate([send_lo, keep_lo]):
            for b in range(B):
                qhalf = jax.lax.dot(
                    xp[b, pl.ds(base, 256), :], wq,
                    preferred_element_type=jnp.float32).astype(jnp.bfloat16)
                for j in range(2):
                    grow = base + GRP * j
                    gl = GRP * j
                    lvecs = []
                    for h in range(Hq):
                        hlo = h * Dh
                        qh = qhalf[gl:gl + GRP, hlo:hlo + Dh]
                        kh = kp[b, pl.ds(grow, GRP), hlo:hlo + Dh]
                        s = jax.lax.dot_general(
                            qh, kh, (((1,), (1,)), ((), ())),
                            preferred_element_type=jnp.float32)
                        w = jnp.exp(s * 0.125)
                        lvecs.append(jnp.sum(w, axis=1))
                        o_acc[b, pl.ds(grow, GRP), hlo:hlo + Dh] = jax.lax.dot(
                            w.astype(jnp.bfloat16),
                            vp[b, pl.ds(grow, GRP), hlo:hlo + Dh],
                            preferred_element_type=jnp.float32,
                        ).astype(jnp.bfloat16)
                    lg = jnp.stack(lvecs, axis=-1)
                    o_acc[b, pl.ds(grow, GRP), HD:HD + Hq] = lg.astype(
                        jnp.bfloat16)
            if p == 0:
                pl.semaphore_wait(barrier_sem, NYZ)
                rox.start()

        rox.wait()
        o_acc[:, pl.ds(keep_lo, 256), :] = (
            o_acc[:, pl.ds(keep_lo, 256), :].astype(jnp.float32)
            + orx[:, :, :].astype(jnp.float32)).astype(jnp.bfloat16)
        lo = keep_lo

        rs = []
        for d in range(1, NYZ):
            pt = (t + d) & 15
            ro = pltpu.make_async_remote_copy(
                src_ref=o_acc.at[:, pl.ds(lo + OWN * pt, OWN), :],
                dst_ref=oryz.at[NYZ - 1 - d],
                send_sem=syz_s.at[d - 1], recv_sem=syz_r.at[NYZ - 1 - d],
                device_id=(yz_peer(d),), device_id_type=pl.DeviceIdType.MESH)
            ro.start()
            rs.append(ro)
        for ro in rs:
            ro.wait()
        lo = lo + OWN * t
        osum = oryz[0, :, :, :].astype(jnp.float32)
        for s in range(1, NYZ - 1):
            osum = osum + oryz[s, :, :, :].astype(jnp.float32)
        o_acc[:, pl.ds(lo, OWN), :] = (
            o_acc[:, pl.ds(lo, OWN), :].astype(jnp.float32)
            + osum).astype(jnp.bfloat16)

        for b in range(B):
            lb = o_acc[b, pl.ds(lo, OWN), HD:HD + Hq].astype(jnp.float32)
            lfull = jnp.broadcast_to(
                lb[:, :, None], (OWN, Hq, Dh)).reshape(OWN, HD)
            o_final[b, pl.ds(lo, OWN), :] = (
                o_acc[b, pl.ds(lo, OWN), 0:HD].astype(jnp.float32)
                / lfull).astype(jnp.bfloat16)

        ags = []
        for d in range(1, NYZ):
            ag = pltpu.make_async_remote_copy(
                src_ref=o_final.at[:, pl.ds(lo, OWN), :],
                dst_ref=o_final.at[:, pl.ds(lo, OWN), :],
                send_sem=agyz_s.at[d - 1], recv_sem=agyz_r.at[NYZ - 1 - d],
                device_id=(yz_peer(d),), device_id_type=pl.DeviceIdType.MESH)
            ag.start()
            ags.append(ag)
        for ag in ags:
            ag.wait_send()
        for s in range(NYZ - 1):
            rr = pltpu.make_async_remote_copy(
                src_ref=o_final.at[:, pl.ds(lo, OWN), :],
                dst_ref=o_final.at[:, pl.ds(lo, OWN), :],
                send_sem=agyz_s.at[s], recv_sem=agyz_r.at[s],
                device_id=(my,), device_id_type=pl.DeviceIdType.MESH)
            rr.wait_recv()
        lo = lo - OWN * t

        agx = pltpu.make_async_remote_copy(
            src_ref=o_final.at[:, pl.ds(lo, 256), :],
            dst_ref=o_final.at[:, pl.ds(lo, 256), :],
            send_sem=agx_s.at[0], recv_sem=agx_r.at[0],
            device_id=(xpartner,), device_id_type=pl.DeviceIdType.MESH)
        agx.start()

        wo = wo_ref[...].astype(jnp.bfloat16)

        def wo_half(half_lo):
            for b in range(B):
                outp = jax.lax.dot(
                    o_final[b, pl.ds(half_lo, 256), :], wo,
                    preferred_element_type=jnp.float32)
                for c in range(4):
                    g = (half_lo + 64 * c) >> 7
                    dst = 64 * g + 256 * (c & 1)
                    out_ref[b, pl.ds(dst, 64), :] = outp[64 * c:64 * c + 64, :]

        wo_half(lo)
        for p in peers[1:]:
            pl.semaphore_signal(exit_sem, inc=1, device_id=(p,),
                                device_id_type=pl.DeviceIdType.MESH)
        agx.wait()
        wo_half(256 - lo)

        pl.semaphore_signal(exit_sem, inc=1, device_id=(xpartner,),
                            device_id_type=pl.DeviceIdType.MESH)
        pl.semaphore_wait(exit_sem, NYZ)

    return pl.pallas_call(
        body,
        out_shape=jax.ShapeDtypeStruct((B, Sq, DMODEL), jnp.float32),
        in_specs=[pl.BlockSpec(memory_space=pltpu.VMEM)] * 5,
        out_specs=pl.BlockSpec(memory_space=pltpu.VMEM),
        scratch_shapes=[
            pltpu.VMEM((B, Sq, HDL), jnp.bfloat16),
            pltpu.VMEM((B, Sq, DMODEL), jnp.bfloat16),
            pltpu.VMEM((B, SKV, HD), jnp.bfloat16),
            pltpu.VMEM((B, SKV, HD), jnp.bfloat16),
            pltpu.VMEM((B, 256, HDL), jnp.bfloat16),
            pltpu.VMEM((NYZ - 1, B, OWN, HDL), jnp.bfloat16),
            pltpu.SemaphoreType.DMA((1,)),
            pltpu.SemaphoreType.DMA((1,)),
            pltpu.SemaphoreType.DMA((NYZ - 1,)),
            pltpu.SemaphoreType.DMA((NYZ - 1,)),
            pltpu.SemaphoreType.DMA((NYZ - 1,)),
            pltpu.SemaphoreType.DMA((NYZ - 1,)),
            pltpu.SemaphoreType.DMA((1,)),
            pltpu.SemaphoreType.DMA((1,)),
            pltpu.VMEM((B, Sq, HD), jnp.bfloat16),
            pltpu.SemaphoreType.REGULAR,
        ],
        compiler_params=pltpu.CompilerParams(collective_id=0),
    )(x, Wq, K2, V2, Wo)
